# baseline (speedup 1.0000x reference)
import functools

import jax
import jax.numpy as jnp
from jax import lax
from jax.experimental import pallas as pl
from jax.experimental.pallas import tpu as pltpu
from jax.experimental.pallas import tpu_sc as plsc

_NBG = 3
_NBT = 2
_NB = 4096
_NS = 200
_D = 32
_CHUNK = 128
_NCHUNKS = _NS
_QROW = 128


def _build_gather():
    info = plsc.get_sparse_core_info()
    NC, NS_sub = info.num_cores, info.num_subcores
    NW = NC * NS_sub
    b_per_w = (_NB // NW) * _NS
    mesh = plsc.VectorSubcoreMesh(core_axis_name="c", subcore_axis_name="s")

    @functools.partial(
        pl.kernel,
        mesh=mesh,
        out_type=jax.ShapeDtypeStruct((_NS, _D // 8, NW, 1024), jnp.float32),
        compiler_params=pltpu.CompilerParams(use_tc_tiling_on_sc=False,
                                             needs_layout_passes=False),
        scratch_types=(
            [pltpu.VMEM((b_per_w,), jnp.int32),
             pltpu.VMEM((_NBG, _CHUNK), jnp.int32),
             pltpu.VMEM((_NBG, _CHUNK), jnp.int32),
             pltpu.VMEM((_NBG, _CHUNK, _QROW), jnp.float32),
             pltpu.VMEM((_NBT, 1, _D // 8, 1, 1024), jnp.float32),
             pltpu.SemaphoreType.DMA((_NBG,)),
             pltpu.SemaphoreType.DMA((_NBT,))]
        ),
    )
    def gather_kernel(table4_hbm, idx_hbm, out_hbm, idx_t, i4_v, sub_v,
                      rows_v, tile_v, gsem, osem):
        wid = lax.axis_index("s") * NC + lax.axis_index("c")
        iota = lax.broadcasted_iota(jnp.int32, (16,), 0)

        pltpu.sync_copy(idx_hbm.at[wid], idx_t)

        def gather_start(c, b):
            coff = pl.multiple_of(c * _CHUNK, 8)
            for k in range(_CHUNK // 16):
                v = idx_t[pl.ds(coff + 16 * k, 16)]
                i4_v[b, pl.ds(16 * k, 16)] = lax.shift_right_logical(v, 1)
                sub_v[b, pl.ds(16 * k, 16)] = lax.shift_left(
                    lax.bitwise_and(v, 1), 5)
            pltpu.async_copy(
                table4_hbm.at[i4_v.at[b]], rows_v.at[b], gsem.at[b])

        def gather_wait(b):
            pltpu.make_async_copy(
                table4_hbm.at[i4_v.at[0]],
                rows_v.at[b], gsem.at[b]).wait()

        def out_start(c, b):
            pltpu.async_copy(
                tile_v.at[b],
                out_hbm.at[pl.ds(c, 1), :, pl.ds(wid, 1)],
                osem.at[b])

        def out_wait(b):
            pltpu.make_async_copy(
                tile_v.at[b],
                out_hbm.at[pl.ds(0, 1), :, pl.ds(0, 1)],
                osem.at[b]).wait()

        rv = [g * 16 + iota for g in range(8)]

        def transpose(bg, bt):
            rows_b = rows_v.at[bg]
            for g in range(8):
                sub16 = sub_v[bg, pl.ds(16 * g, 16)]
                for dr in range(_D // 8):
                    for r in range(8):
                        d = dr * 8 + r
                        v = plsc.load_gather(rows_b, [rv[g], sub16 + d])
                        tile_v[bt, 0, dr, 0,
                               pl.ds(r * 128 + 16 * g, 16)] = v

        for b in range(_NBG):
            gather_start(b, b)

        def chunk_body(c, carry):
            bg = lax.rem(c, _NBG)
            bt = lax.rem(c, _NBT)
            gather_wait(bg)
            @pl.when(c >= _NBT)
            def _():
                out_wait(bt)
            transpose(bg, bt)
            @pl.when(c + _NBG < _NCHUNKS)
            def _():
                gather_start(c + _NBG, bg)
            out_start(c, bt)
            return carry

        lax.fori_loop(0, _NCHUNKS, chunk_body, 0)

        for b in range(_NBT):
            out_wait(b)

    return gather_kernel


@jax.jit
def kernel(indices, table):
    idx_w = (indices.astype(jnp.int32).T
             .reshape(_NS, _NB // 128, 128)
             .transpose(1, 0, 2)
             .reshape(_NB // 128, _NS * 128))
    tbl = jnp.pad(table.reshape(500000, 64), ((0, 0), (0, 64)))
    out4 = _build_gather()(tbl, idx_w)
    out = out4.reshape(_NS, _D // 8, 32, 8, 128)
    out = out.transpose(2, 4, 0, 1, 3).reshape(_NB, _NS, _D)
    return out


# speedup vs baseline: 1.2031x; 1.2031x over previous
import functools

import jax
import jax.numpy as jnp
from jax import lax
from jax.experimental import pallas as pl
from jax.experimental.pallas import tpu as pltpu
from jax.experimental.pallas import tpu_sc as plsc

_NBG = 3
_NBT = 2
_NB = 4096
_NS = 200
_D = 32
_SG = 4
_CHUNK = _SG * 128
_NCHUNKS = _NS // _SG


def _build_gather():
    info = plsc.get_sparse_core_info()
    NC, NS_sub = info.num_cores, info.num_subcores
    NW = NC * NS_sub
    b_per_w = (_NB // NW) * _NS
    mesh = plsc.VectorSubcoreMesh(core_axis_name="c", subcore_axis_name="s")

    @functools.partial(
        pl.kernel,
        mesh=mesh,
        out_type=jax.ShapeDtypeStruct((_NS, _D // 8, NW, 1024), jnp.float32),
        compiler_params=pltpu.CompilerParams(use_tc_tiling_on_sc=False,
                                             needs_layout_passes=False),
        scratch_types=(
            [pltpu.VMEM((b_per_w,), jnp.int32),
             pltpu.VMEM((_NBG, _CHUNK, _D), jnp.float32),
             pltpu.VMEM((_NBT, _SG, _D // 8, 1, 1024), jnp.float32),
             pltpu.SemaphoreType.DMA((_NBG,)),
             pltpu.SemaphoreType.DMA((_NBT,))]
        ),
    )
    def gather_kernel(table_hbm, idx_hbm, out_hbm, idx_t,
                      rows_v, tile_v, gsem, osem):
        wid = lax.axis_index("s") * NC + lax.axis_index("c")
        iota = lax.broadcasted_iota(jnp.int32, (16,), 0)

        pltpu.sync_copy(idx_hbm.at[wid], idx_t)

        def gather_start(c, b):
            coff = pl.multiple_of(c * _CHUNK, 8)
            pltpu.async_copy(
                table_hbm.at[idx_t.at[pl.ds(coff, _CHUNK)]],
                rows_v.at[b], gsem.at[b])

        def gather_wait(b):
            pltpu.make_async_copy(
                table_hbm.at[idx_t.at[pl.ds(0, _CHUNK)]],
                rows_v.at[b], gsem.at[b]).wait()

        def out_start(c, b):
            pltpu.async_copy(
                tile_v.at[b],
                out_hbm.at[pl.ds(c * _SG, _SG), :, pl.ds(wid, 1)],
                osem.at[b])

        def out_wait(b):
            pltpu.make_async_copy(
                tile_v.at[b],
                out_hbm.at[pl.ds(0, _SG), :, pl.ds(0, 1)],
                osem.at[b]).wait()

        rv = [[sl * 128 + g * 16 + iota for g in range(8)]
              for sl in range(_SG)]
        cv = [iota * 0 + d for d in range(_D)]

        def transpose(bg, bt):
            rows_b = rows_v.at[bg]
            for sl in range(_SG):
                for dr in range(_D // 8):
                    for r in range(8):
                        d = dr * 8 + r
                        for g in range(8):
                            v = plsc.load_gather(rows_b, [rv[sl][g], cv[d]])
                            tile_v[bt, sl, dr, 0,
                                   pl.ds(r * 128 + 16 * g, 16)] = v

        for b in range(_NBG):
            gather_start(b, b)

        def chunk_body(c, carry):
            bg = lax.rem(c, _NBG)
            bt = lax.rem(c, _NBT)
            gather_wait(bg)
            @pl.when(c >= _NBT)
            def _():
                out_wait(bt)
            transpose(bg, bt)
            @pl.when(c + _NBG < _NCHUNKS)
            def _():
                gather_start(c + _NBG, bg)
            out_start(c, bt)
            return carry

        lax.fori_loop(0, _NCHUNKS, chunk_body, 0)

        for b in range(_NBT):
            out_wait(b)

    return gather_kernel


@jax.jit
def kernel(indices, table):
    idx_w = (indices.astype(jnp.int32).T
             .reshape(_NS, _NB // 128, 128)
             .transpose(1, 0, 2)
             .reshape(_NB // 128, _NS * 128))
    out4 = _build_gather()(table, idx_w)
    out = out4.reshape(_NS, _D // 8, 32, 8, 128)
    out = out.transpose(2, 4, 0, 1, 3).reshape(_NB, _NS, _D)
    return out
